# baseline (speedup 1.0000x reference)
import functools

import jax
import jax.numpy as jnp
from jax import lax
from jax.experimental import pallas as pl
from jax.experimental.pallas import tpu as pltpu
from jax.experimental.pallas import tpu_sc as plsc

N = 20000
IMG_W = 1333.0
IMG_H = 800.0
SCORE_THRESH = 0.5
NMS_THRESH = 0.5
TOPK_PER_IMAGE = 100

NC = 2048
SEL = 2000
BLK = 128
NBLK = NC // BLK

NCH = N // 16
NBIN = 4096
INVALID = 0x00FFFFFF

_sc_mesh = plsc.VectorSubcoreMesh(core_axis_name="c", subcore_axis_name="s",
                                  num_cores=2, num_subcores=16)

_I16 = lambda: lax.iota(jnp.int32, 16)


def _scan_hist(hist_v, target):
    def body(i, carry):
        b, nbefore, acc = carry
        h = hist_v[pl.ds(i * 16, 16)]
        c = plsc.cumsum(h) + acc
        hit = (c >= target) & (b < 0)
        lane = jnp.max(plsc.all_reduce_ffs(hit))
        found_now = lane < 16
        nb_cand = jnp.max(jnp.where(_I16() == lane, c - h, 0))
        b_new = jnp.where((b < 0) & found_now, i * 16 + lane, b)
        nb_new = jnp.where((b < 0) & found_now, nb_cand, nbefore)
        return b_new, nb_new, jnp.max(c)
    b, nbefore, _ = plsc.parallel_loop(
        0, NBIN // 16, 1, unroll=8,
        carry=(jnp.int32(-1), jnp.int32(0), jnp.int32(0)))(body)
    return b, nbefore


def _zero(ref, n):
    def body(i):
        ref[pl.ds(i * 16, 16)] = jnp.zeros((16,), jnp.int32)
    plsc.parallel_loop(0, n // 16, 1, unroll=8)(body)


@functools.partial(
    pl.kernel,
    out_type=(
        jax.ShapeDtypeStruct((NC,), jnp.float32),
        jax.ShapeDtypeStruct((NC,), jnp.int32),
        jax.ShapeDtypeStruct((NC,), jnp.float32),
        jax.ShapeDtypeStruct((NC,), jnp.float32),
        jax.ShapeDtypeStruct((NC,), jnp.float32),
        jax.ShapeDtypeStruct((NC,), jnp.float32),
    ),
    mesh=_sc_mesh,
    scratch_types=[
        pltpu.VMEM((N,), jnp.float32),
        pltpu.VMEM((N,), jnp.int32),
        pltpu.VMEM((NBIN,), jnp.int32),
        pltpu.VMEM((NC,), jnp.int32),
        pltpu.VMEM((NC,), jnp.int32),
        pltpu.VMEM((NC,), jnp.int32),
        pltpu.VMEM((NC,), jnp.int32),
        pltpu.VMEM((NC,), jnp.float32),
        pltpu.VMEM((NC,), jnp.float32),
    ],
    compiler_params=pltpu.CompilerParams(needs_layout_passes=False,
                                         use_tc_tiling_on_sc=False),
)
def _select_sort(s_hbm, x1_hbm, y1_hbm, x2_hbm, y2_hbm,
                 score_out, idx_out, x1_out, y1_out, x2_out, y2_out,
                 s_v, dkey_v, hist_v, ka_v, va_v, kb_v, vb_v,
                 score_v, coord_v):
    wid = lax.axis_index("s") * 2 + lax.axis_index("c")

    @pl.when(wid == 0)
    def _():
        pltpu.sync_copy(s_hbm, s_v)

        _zero(hist_v, NBIN)

        def p1(i):
            sv = s_v[pl.ds(i * 16, 16)]
            bits = plsc.bitcast(sv, jnp.int32)
            m = bits & 0x7FFFFF
            dk = jnp.where(sv > SCORE_THRESH, 0x7FFFFF - m, INVALID)
            dkey_v[pl.ds(i * 16, 16)] = dk
            d1 = dk >> 12
            cnt, lastm = plsc.scan_count(d1)
            plsc.addupdate_scatter(hist_v, [d1], cnt, mask=lastm)
        plsc.parallel_loop(0, NCH, 1, unroll=8)(p1)

        b1, nbefore1 = _scan_hist(hist_v, SEL)

        _zero(hist_v, NBIN)

        def p2(i):
            dk = dkey_v[pl.ds(i * 16, 16)]
            sel = (dk >> 12) == b1
            d2 = dk & 0xFFF
            cnt, lastm = plsc.scan_count(d2, mask=sel)
            plsc.addupdate_scatter(hist_v, [d2], cnt, mask=lastm)
        plsc.parallel_loop(0, NCH, 1, unroll=8)(p2)

        b2, nbefore2 = _scan_hist(hist_v, SEL - nbefore1)
        cutkey = (b1 << 12) | b2
        need_eq = SEL - (nbefore1 + nbefore2)

        def fill(i):
            ka_v[pl.ds(i * 16, 16)] = jnp.full((16,), INVALID, jnp.int32)
            va_v[pl.ds(i * 16, 16)] = jnp.zeros((16,), jnp.int32)
        plsc.parallel_loop(0, NC // 16, 1, unroll=8)(fill)

        def p3(i, carry):
            nsel, neq = carry
            dk = dkey_v[pl.ds(i * 16, 16)]
            lt = dk < cutkey
            eq = dk == cutkey
            eqi = jnp.asarray(eq, jnp.int32)
            excl_eq = plsc.cumsum(eqi) - eqi
            sel = lt | (eq & ((neq + excl_eq) < need_eq))
            seli = jnp.asarray(sel, jnp.int32)
            pos = nsel + plsc.cumsum(seli) - seli
            idxv = i * 16 + _I16()
            plsc.store_scatter(ka_v, [pos], dk, mask=sel)
            plsc.store_scatter(va_v, [pos], idxv, mask=sel)
            return nsel + jnp.sum(seli), neq + jnp.sum(eqi)
        plsc.parallel_loop(0, NCH, 1, unroll=8,
                           carry=(jnp.int32(0), jnp.int32(0)))(p3)

        for shift, src_k, src_v, dst_k, dst_v in (
                (0, ka_v, va_v, kb_v, vb_v),
                (12, kb_v, vb_v, ka_v, va_v)):
            _zero(hist_v, NBIN)

            def ph(i, shift=shift, src_k=src_k):
                d = (src_k[pl.ds(i * 16, 16)] >> shift) & 0xFFF
                cnt, lastm = plsc.scan_count(d)
                plsc.addupdate_scatter(hist_v, [d], cnt, mask=lastm)
            plsc.parallel_loop(0, NC // 16, 1, unroll=8)(ph)

            def pp(i, acc):
                h = hist_v[pl.ds(i * 16, 16)]
                hist_v[pl.ds(i * 16, 16)] = acc + plsc.cumsum(h) - h
                return acc + jnp.sum(h)
            plsc.parallel_loop(0, NBIN // 16, 1, unroll=8,
                               carry=jnp.int32(0))(pp)

            def pm(i, _, shift=shift, src_k=src_k, src_v=src_v,
                   dst_k=dst_k, dst_v=dst_v):
                k = src_k[pl.ds(i * 16, 16)]
                v = src_v[pl.ds(i * 16, 16)]
                d = (k >> shift) & 0xFFF
                base = plsc.load_gather(hist_v, [d])
                cnt, lastm = plsc.scan_count(d)
                pos = base + cnt - 1
                plsc.store_scatter(dst_k, [pos], k)
                plsc.store_scatter(dst_v, [pos], v)
                plsc.addupdate_scatter(hist_v, [d], cnt, mask=lastm)
                return 0
            lax.fori_loop(0, NC // 16, pm, 0)

        def p5(i):
            dk = ka_v[pl.ds(i * 16, 16)]
            valid = dk < INVALID
            sbits = 0x3F000000 | (0x7FFFFF - dk)
            sv = jnp.where(valid, plsc.bitcast(sbits, jnp.float32), -1.0)
            score_v[pl.ds(i * 16, 16)] = sv
        plsc.parallel_loop(0, NC // 16, 1, unroll=8)(p5)

        pltpu.sync_copy(score_v, score_out)
        pltpu.sync_copy(va_v, idx_out)

        for src, dst in ((x1_hbm, x1_out), (y1_hbm, y1_out),
                         (x2_hbm, x2_out), (y2_hbm, y2_out)):
            pltpu.sync_copy(src, s_v)

            def pg(i):
                idc = va_v[pl.ds(i * 16, 16)]
                coord_v[pl.ds(i * 16, 16)] = plsc.load_gather(s_v, [idc])
            plsc.parallel_loop(0, NC // 16, 1, unroll=8)(pg)
            pltpu.sync_copy(coord_v, dst)


def _nms_kernel(x1c_ref, y1c_ref, x2c_ref, y2c_ref,
                x1r_ref, y1r_ref, x2r_ref, y2r_ref,
                valid_ref, keep_out_ref, keep_ref, t_ref):
    keep_ref[:, :] = valid_ref[:, :]

    lane = lax.broadcasted_iota(jnp.int32, (1, BLK), 1)
    rowid = lax.broadcasted_iota(jnp.int32, (BLK, BLK), 0)
    colid = lax.broadcasted_iota(jnp.int32, (BLK, BLK), 1)
    upper = (colid > rowid).astype(jnp.float32)
    lower = (colid < rowid).astype(jnp.float32)

    for bi in range(NBLK):
        base = bi * BLK
        w = NC - base
        x1c = x1c_ref[pl.ds(base, BLK), :]
        y1c = y1c_ref[pl.ds(base, BLK), :]
        x2c = x2c_ref[pl.ds(base, BLK), :]
        y2c = y2c_ref[pl.ds(base, BLK), :]
        area_c = jnp.clip(x2c - x1c, 0.0) * jnp.clip(y2c - y1c, 0.0)

        x1r = x1r_ref[:, base:]
        y1r = y1r_ref[:, base:]
        x2r = x2r_ref[:, base:]
        y2r = y2r_ref[:, base:]
        area_r = jnp.clip(x2r - x1r, 0.0) * jnp.clip(y2r - y1r, 0.0)

        ltx = jnp.maximum(x1c, x1r)
        lty = jnp.maximum(y1c, y1r)
        rbx = jnp.minimum(x2c, x2r)
        rby = jnp.minimum(y2c, y2r)
        wx = jnp.clip(rbx - ltx, 0.0)
        wy = jnp.clip(rby - lty, 0.0)
        inter = wx * wy
        iou = inter / (area_c + area_r - inter + 1e-9)
        over = (iou > NMS_THRESH).astype(jnp.float32)

        tdiag = over[:, :BLK]
        t_ref[:, :] = tdiag * upper
        rowact = jnp.max(tdiag * lower, axis=0, keepdims=True)

        keep_blk = keep_ref[:, base:base + BLK]
        pending = keep_blk * rowact

        def nxt(pend):
            return jnp.min(jnp.where(pend > 0.0, lane, jnp.int32(9999)))

        def cond(state):
            return state[2] < 9999

        def body(state):
            kb, pend, i = state
            row = t_ref[pl.ds(i, 1), :]
            kb = kb * (1.0 - row)
            pend = pend * (1.0 - row)
            pend = jnp.where(lane == i, 0.0, pend)
            return kb, pend, nxt(pend)

        keep_blk, _, _ = lax.while_loop(
            cond, body, (keep_blk, pending, nxt(pending)))
        keep_ref[:, base:base + BLK] = keep_blk

        if bi + 1 < NBLK:
            kcol = keep_blk.reshape(BLK, 1)
            sup = jnp.max(over[:, BLK:] * kcol, axis=0, keepdims=True)
            keep_ref[:, base + BLK:] = keep_ref[:, base + BLK:] * (1.0 - sup)

    keep_out_ref[:, :] = keep_ref[:, :]


def _run_nms(b, valid_f):
    cols = [b[:, k:k + 1] for k in range(4)]
    rows = [b[:, k].reshape(1, NC) for k in range(4)]
    keep = pl.pallas_call(
        _nms_kernel,
        out_shape=jax.ShapeDtypeStruct((1, NC), jnp.float32),
        scratch_shapes=[
            pltpu.VMEM((1, NC), jnp.float32),
            pltpu.VMEM((BLK, BLK), jnp.float32),
        ],
    )(*cols, *rows, valid_f.reshape(1, NC))
    return keep[0]


def _delta_to_bases(b6, boxes):
    x1 = boxes[:, 0]; y1 = boxes[:, 1]; x2 = boxes[:, 2]; y2 = boxes[:, 3]
    dx = x2 - x1
    dy = y2 - y1
    midx = (x1 + x2) / 2.0 + b6[:, 0] * dx
    midy = (y1 + y2) / 2.0 + b6[:, 1] * dy
    firstx = b6[:, 2]; firsty = b6[:, 3]; secondx = b6[:, 4]; secondy = b6[:, 5]
    X1 = midx + firstx * dx
    Y1 = midy + firsty * dy
    X2 = midx + secondx * dx
    Y2 = midy + secondy * dy
    X3 = midx - secondx * dx
    Y3 = midy - secondy * dy
    X4 = midx - firstx * dx
    Y4 = midy - firsty * dy
    return jnp.stack((X1, Y1, X2, Y2, X3, Y3, X4, Y4, midx, midy), axis=-1)


FIN = 128


@functools.partial(
    pl.kernel,
    out_type=jax.ShapeDtypeStruct((16 * FIN,), jnp.float32),
    mesh=_sc_mesh,
    scratch_types=[
        pltpu.VMEM((NC,), jnp.float32),
        pltpu.VMEM((NC,), jnp.float32),
        pltpu.VMEM((NC,), jnp.int32),
        pltpu.VMEM((NC,), jnp.float32),
        pltpu.VMEM((NC,), jnp.float32),
        pltpu.VMEM((NC,), jnp.float32),
        pltpu.VMEM((NC,), jnp.float32),
        pltpu.VMEM((NC,), jnp.int32),
        pltpu.VMEM((N,), jnp.float32),
        pltpu.VMEM((FIN,), jnp.int32),
        pltpu.VMEM((7 * FIN,), jnp.float32),
        pltpu.VMEM((16 * FIN,), jnp.float32),
    ],
    compiler_params=pltpu.CompilerParams(needs_layout_passes=False,
                                         use_tc_tiling_on_sc=False),
)
def _finalize(keep_hbm, score_hbm, idx_hbm, x1_hbm, y1_hbm, x2_hbm, y2_hbm,
              b0_hbm, b1_hbm, b2_hbm, b3_hbm, b4_hbm, b5_hbm, b6_hbm,
              out_hbm,
              keep_v, fscore_v, fidx_v, fx1_v, fy1_v, fx2_v, fy2_v,
              order_v, tab_v, oi_v, bas_v, out_v):
    wid = lax.axis_index("s") * 2 + lax.axis_index("c")

    @pl.when(wid == 0)
    def _():
        pltpu.sync_copy(keep_hbm, keep_v)
        pltpu.sync_copy(score_hbm, fscore_v)
        pltpu.sync_copy(idx_hbm, fidx_v)
        pltpu.sync_copy(x1_hbm, fx1_v)
        pltpu.sync_copy(y1_hbm, fy1_v)
        pltpu.sync_copy(x2_hbm, fx2_v)
        pltpu.sync_copy(y2_hbm, fy2_v)

        def c1(i, acc):
            k = keep_v[pl.ds(i * 16, 16)] > 0.5
            return acc + jnp.sum(jnp.asarray(k, jnp.int32))
        K = plsc.parallel_loop(0, NC // 16, 1, unroll=8,
                               carry=jnp.int32(0))(c1)

        def c2(i, carry):
            nk, nu = carry
            k = keep_v[pl.ds(i * 16, 16)] > 0.5
            ki = jnp.asarray(k, jnp.int32)
            ui = 1 - ki
            posk = nk + plsc.cumsum(ki) - ki
            posu = K + nu + plsc.cumsum(ui) - ui
            cand = i * 16 + _I16()
            plsc.store_scatter(order_v, [posk], cand, mask=k)
            plsc.store_scatter(order_v, [posu], cand, mask=~k)
            return nk + jnp.sum(ki), nu + jnp.sum(ui)
        plsc.parallel_loop(0, NC // 16, 1, unroll=8,
                           carry=(jnp.int32(0), jnp.int32(0)))(c2)

        def c3(j):
            ordc = order_v[pl.ds(j * 16, 16)]
            rows = j * 16 + _I16()
            sc = plsc.load_gather(fscore_v, [ordc])
            out_v[pl.ds(15 * FIN + j * 16, 16)] = jnp.where(
                rows < K, sc, -1.0)
            oi_v[pl.ds(j * 16, 16)] = plsc.load_gather(fidx_v, [ordc])
            x1 = jnp.clip(plsc.load_gather(fx1_v, [ordc]), 0.0, IMG_W)
            y1 = jnp.clip(plsc.load_gather(fy1_v, [ordc]), 0.0, IMG_H)
            x2 = jnp.clip(plsc.load_gather(fx2_v, [ordc]), 0.0, IMG_W)
            y2 = jnp.clip(plsc.load_gather(fy2_v, [ordc]), 0.0, IMG_H)
            out_v[pl.ds(11 * FIN + j * 16, 16)] = x1
            out_v[pl.ds(12 * FIN + j * 16, 16)] = y1
            out_v[pl.ds(13 * FIN + j * 16, 16)] = x2
            out_v[pl.ds(14 * FIN + j * 16, 16)] = y2
        plsc.parallel_loop(0, FIN // 16, 1, unroll=4)(c3)

        for c, src in enumerate((b0_hbm, b1_hbm, b2_hbm, b3_hbm,
                                 b4_hbm, b5_hbm, b6_hbm)):
            pltpu.sync_copy(src, tab_v)

            def c4(j, c=c):
                oic = oi_v[pl.ds(j * 16, 16)]
                bas_v[pl.ds(c * FIN + j * 16, 16)] = plsc.load_gather(
                    tab_v, [oic])
            plsc.parallel_loop(0, FIN // 16, 1, unroll=4)(c4)

        def c5(j):
            x1 = out_v[pl.ds(11 * FIN + j * 16, 16)]
            y1 = out_v[pl.ds(12 * FIN + j * 16, 16)]
            x2 = out_v[pl.ds(13 * FIN + j * 16, 16)]
            y2 = out_v[pl.ds(14 * FIN + j * 16, 16)]
            b0 = bas_v[pl.ds(0 * FIN + j * 16, 16)]
            b1 = bas_v[pl.ds(1 * FIN + j * 16, 16)]
            b2 = bas_v[pl.ds(2 * FIN + j * 16, 16)]
            b3 = bas_v[pl.ds(3 * FIN + j * 16, 16)]
            b4 = bas_v[pl.ds(4 * FIN + j * 16, 16)]
            b5 = bas_v[pl.ds(5 * FIN + j * 16, 16)]
            b6 = bas_v[pl.ds(6 * FIN + j * 16, 16)]
            dx = x2 - x1
            dy = y2 - y1
            midx = (x1 + x2) / 2.0 + b0 * dx
            midy = (y1 + y2) / 2.0 + b1 * dy
            out_v[pl.ds(0 * FIN + j * 16, 16)] = midx + b2 * dx
            out_v[pl.ds(1 * FIN + j * 16, 16)] = midy + b3 * dy
            out_v[pl.ds(2 * FIN + j * 16, 16)] = midx + b4 * dx
            out_v[pl.ds(3 * FIN + j * 16, 16)] = midy + b5 * dy
            out_v[pl.ds(4 * FIN + j * 16, 16)] = midx - b4 * dx
            out_v[pl.ds(5 * FIN + j * 16, 16)] = midy - b5 * dy
            out_v[pl.ds(6 * FIN + j * 16, 16)] = midx - b2 * dx
            out_v[pl.ds(7 * FIN + j * 16, 16)] = midy - b3 * dy
            out_v[pl.ds(8 * FIN + j * 16, 16)] = midx
            out_v[pl.ds(9 * FIN + j * 16, 16)] = midy
            out_v[pl.ds(10 * FIN + j * 16, 16)] = dy + b6 * dy
        plsc.parallel_loop(0, FIN // 16, 1, unroll=4)(c5)

        pltpu.sync_copy(out_v, out_hbm)


def kernel(boxes, scores, bases):
    s = scores[:, 0]
    c_sc, c_idx, cx1, cy1, cx2, cy2 = _select_sort(
        s, boxes[:, 0], boxes[:, 1], boxes[:, 2], boxes[:, 3])
    bx = jnp.stack([
        jnp.clip(cx1, 0.0, IMG_W),
        jnp.clip(cy1, 0.0, IMG_H),
        jnp.clip(cx2, 0.0, IMG_W),
        jnp.clip(cy2, 0.0, IMG_H),
    ], axis=1)
    valid = c_sc > SCORE_THRESH

    keep_f = _run_nms(bx, valid.astype(jnp.float32))

    out_flat = _finalize(keep_f, c_sc, c_idx, cx1, cy1, cx2, cy2,
                         bases[:, 0], bases[:, 1], bases[:, 2], bases[:, 3],
                         bases[:, 4], bases[:, 5], bases[:, 6])
    return jnp.stack(
        [out_flat[c * FIN:c * FIN + TOPK_PER_IMAGE] for c in range(16)],
        axis=1)


# speedup vs baseline: 1.0127x; 1.0127x over previous
import functools

import jax
import jax.numpy as jnp
from jax import lax
from jax.experimental import pallas as pl
from jax.experimental.pallas import tpu as pltpu
from jax.experimental.pallas import tpu_sc as plsc

N = 20000
IMG_W = 1333.0
IMG_H = 800.0
SCORE_THRESH = 0.5
NMS_THRESH = 0.5
TOPK_PER_IMAGE = 100

NC = 2048
SEL = 2000
BLK = 128
NBLK = NC // BLK

NCH = N // 16
NBIN = 4096
INVALID = 0x00FFFFFF

_sc_mesh = plsc.VectorSubcoreMesh(core_axis_name="c", subcore_axis_name="s",
                                  num_cores=2, num_subcores=16)

_I16 = lambda: lax.iota(jnp.int32, 16)


def _scan_hist(hist_v, target):
    def body(i, carry):
        b, nbefore, acc = carry
        h = hist_v[pl.ds(i * 16, 16)]
        c = plsc.cumsum(h) + acc
        hit = (c >= target) & (b < 0)
        lane = jnp.max(plsc.all_reduce_ffs(hit))
        found_now = lane < 16
        nb_cand = jnp.max(jnp.where(_I16() == lane, c - h, 0))
        b_new = jnp.where((b < 0) & found_now, i * 16 + lane, b)
        nb_new = jnp.where((b < 0) & found_now, nb_cand, nbefore)
        return b_new, nb_new, jnp.max(c)
    b, nbefore, _ = lax.fori_loop(
        0, NBIN // 16, body,
        (jnp.int32(-1), jnp.int32(0), jnp.int32(0)))
    return b, nbefore


def _zero(ref, n):
    def body(i):
        ref[pl.ds(i * 16, 16)] = jnp.zeros((16,), jnp.int32)
    plsc.parallel_loop(0, n // 16, 1, unroll=8)(body)


@functools.partial(
    pl.kernel,
    out_type=(
        jax.ShapeDtypeStruct((NC,), jnp.float32),
        jax.ShapeDtypeStruct((NC,), jnp.int32),
        jax.ShapeDtypeStruct((NC,), jnp.float32),
        jax.ShapeDtypeStruct((NC,), jnp.float32),
        jax.ShapeDtypeStruct((NC,), jnp.float32),
        jax.ShapeDtypeStruct((NC,), jnp.float32),
    ),
    mesh=_sc_mesh,
    scratch_types=[
        pltpu.VMEM((N,), jnp.float32),
        pltpu.VMEM((N,), jnp.int32),
        pltpu.VMEM((NBIN,), jnp.int32),
        pltpu.VMEM((NC,), jnp.int32),
        pltpu.VMEM((NC,), jnp.int32),
        pltpu.VMEM((NC,), jnp.int32),
        pltpu.VMEM((NC,), jnp.int32),
        pltpu.VMEM((NC,), jnp.float32),
        pltpu.VMEM((NC,), jnp.float32),
    ],
    compiler_params=pltpu.CompilerParams(needs_layout_passes=False,
                                         use_tc_tiling_on_sc=False),
)
def _select_sort(s_hbm, x1_hbm, y1_hbm, x2_hbm, y2_hbm,
                 score_out, idx_out, x1_out, y1_out, x2_out, y2_out,
                 s_v, dkey_v, hist_v, ka_v, va_v, kb_v, vb_v,
                 score_v, coord_v):
    wid = lax.axis_index("s") * 2 + lax.axis_index("c")

    @pl.when(wid == 0)
    def _():
        pltpu.sync_copy(s_hbm, s_v)

        _zero(hist_v, NBIN)

        def p1(i):
            sv = s_v[pl.ds(i * 16, 16)]
            bits = plsc.bitcast(sv, jnp.int32)
            m = bits & 0x7FFFFF
            dk = jnp.where(sv > SCORE_THRESH, 0x7FFFFF - m, INVALID)
            dkey_v[pl.ds(i * 16, 16)] = dk
            d1 = dk >> 12
            cnt, lastm = plsc.scan_count(d1)
            plsc.addupdate_scatter(hist_v, [d1], cnt, mask=lastm)
        plsc.parallel_loop(0, NCH, 1, unroll=8)(p1)

        b1, nbefore1 = _scan_hist(hist_v, SEL)

        _zero(hist_v, NBIN)

        def p2(i):
            dk = dkey_v[pl.ds(i * 16, 16)]
            sel = (dk >> 12) == b1
            d2 = dk & 0xFFF
            cnt, lastm = plsc.scan_count(d2, mask=sel)
            plsc.addupdate_scatter(hist_v, [d2], cnt, mask=lastm)
        plsc.parallel_loop(0, NCH, 1, unroll=8)(p2)

        b2, nbefore2 = _scan_hist(hist_v, SEL - nbefore1)
        cutkey = (b1 << 12) | b2
        need_eq = SEL - (nbefore1 + nbefore2)

        def fill(i):
            ka_v[pl.ds(i * 16, 16)] = jnp.full((16,), INVALID, jnp.int32)
            va_v[pl.ds(i * 16, 16)] = jnp.zeros((16,), jnp.int32)
        plsc.parallel_loop(0, NC // 16, 1, unroll=8)(fill)

        def p3(i, carry):
            nsel, neq = carry
            dk = dkey_v[pl.ds(i * 16, 16)]
            lt = dk < cutkey
            eq = dk == cutkey
            eqi = jnp.asarray(eq, jnp.int32)
            excl_eq = plsc.cumsum(eqi) - eqi
            sel = lt | (eq & ((neq + excl_eq) < need_eq))
            seli = jnp.asarray(sel, jnp.int32)
            pos = nsel + plsc.cumsum(seli) - seli
            idxv = i * 16 + _I16()
            plsc.store_scatter(ka_v, [pos], dk, mask=sel)
            plsc.store_scatter(va_v, [pos], idxv, mask=sel)
            return nsel + jnp.sum(seli), neq + jnp.sum(eqi)
        plsc.parallel_loop(0, NCH, 1, unroll=8,
                           carry=(jnp.int32(0), jnp.int32(0)))(p3)

        for shift, src_k, src_v, dst_k, dst_v in (
                (0, ka_v, va_v, kb_v, vb_v),
                (12, kb_v, vb_v, ka_v, va_v)):
            _zero(hist_v, NBIN)

            def ph(i, shift=shift, src_k=src_k):
                d = (src_k[pl.ds(i * 16, 16)] >> shift) & 0xFFF
                cnt, lastm = plsc.scan_count(d)
                plsc.addupdate_scatter(hist_v, [d], cnt, mask=lastm)
            plsc.parallel_loop(0, NC // 16, 1, unroll=8)(ph)

            def pp(i, acc):
                h = hist_v[pl.ds(i * 16, 16)]
                hist_v[pl.ds(i * 16, 16)] = acc + plsc.cumsum(h) - h
                return acc + jnp.sum(h)
            plsc.parallel_loop(0, NBIN // 16, 1, unroll=8,
                               carry=jnp.int32(0))(pp)

            def pm(i, _, shift=shift, src_k=src_k, src_v=src_v,
                   dst_k=dst_k, dst_v=dst_v):
                k = src_k[pl.ds(i * 16, 16)]
                v = src_v[pl.ds(i * 16, 16)]
                d = (k >> shift) & 0xFFF
                base = plsc.load_gather(hist_v, [d])
                cnt, lastm = plsc.scan_count(d)
                pos = base + cnt - 1
                plsc.store_scatter(dst_k, [pos], k)
                plsc.store_scatter(dst_v, [pos], v)
                plsc.addupdate_scatter(hist_v, [d], cnt, mask=lastm)
                return 0
            lax.fori_loop(0, NC // 16, pm, 0)

        def p5(i):
            dk = ka_v[pl.ds(i * 16, 16)]
            valid = dk < INVALID
            sbits = 0x3F000000 | (0x7FFFFF - dk)
            sv = jnp.where(valid, plsc.bitcast(sbits, jnp.float32), -1.0)
            score_v[pl.ds(i * 16, 16)] = sv
        plsc.parallel_loop(0, NC // 16, 1, unroll=8)(p5)

        pltpu.sync_copy(score_v, score_out)
        pltpu.sync_copy(va_v, idx_out)

        for src, dst in ((x1_hbm, x1_out), (y1_hbm, y1_out),
                         (x2_hbm, x2_out), (y2_hbm, y2_out)):
            pltpu.sync_copy(src, s_v)

            def pg(i):
                idc = va_v[pl.ds(i * 16, 16)]
                coord_v[pl.ds(i * 16, 16)] = plsc.load_gather(s_v, [idc])
            plsc.parallel_loop(0, NC // 16, 1, unroll=8)(pg)
            pltpu.sync_copy(coord_v, dst)


def _nms_kernel(x1c_ref, y1c_ref, x2c_ref, y2c_ref,
                x1r_ref, y1r_ref, x2r_ref, y2r_ref,
                valid_ref, keep_out_ref, keep_ref, t_ref):
    keep_ref[:, :] = valid_ref[:, :]

    lane = lax.broadcasted_iota(jnp.int32, (1, BLK), 1)
    rowid = lax.broadcasted_iota(jnp.int32, (BLK, BLK), 0)
    colid = lax.broadcasted_iota(jnp.int32, (BLK, BLK), 1)
    upper = (colid > rowid).astype(jnp.float32)
    lower = (colid < rowid).astype(jnp.float32)

    for bi in range(NBLK):
        base = bi * BLK
        w = NC - base
        x1c = x1c_ref[pl.ds(base, BLK), :]
        y1c = y1c_ref[pl.ds(base, BLK), :]
        x2c = x2c_ref[pl.ds(base, BLK), :]
        y2c = y2c_ref[pl.ds(base, BLK), :]
        area_c = jnp.clip(x2c - x1c, 0.0) * jnp.clip(y2c - y1c, 0.0)

        x1r = x1r_ref[:, base:]
        y1r = y1r_ref[:, base:]
        x2r = x2r_ref[:, base:]
        y2r = y2r_ref[:, base:]
        area_r = jnp.clip(x2r - x1r, 0.0) * jnp.clip(y2r - y1r, 0.0)

        ltx = jnp.maximum(x1c, x1r)
        lty = jnp.maximum(y1c, y1r)
        rbx = jnp.minimum(x2c, x2r)
        rby = jnp.minimum(y2c, y2r)
        wx = jnp.clip(rbx - ltx, 0.0)
        wy = jnp.clip(rby - lty, 0.0)
        inter = wx * wy
        iou = inter / (area_c + area_r - inter + 1e-9)
        over = (iou > NMS_THRESH).astype(jnp.float32)

        tdiag = over[:, :BLK]
        t_ref[:, :] = tdiag * upper
        rowact = jnp.max(tdiag * lower, axis=0, keepdims=True)

        keep_blk = keep_ref[:, base:base + BLK]
        pending = keep_blk * rowact

        def nxt(pend):
            return jnp.min(jnp.where(pend > 0.0, lane, jnp.int32(9999)))

        def cond(state):
            return state[2] < 9999

        def body(state):
            kb, pend, i = state
            row = t_ref[pl.ds(i, 1), :]
            kb = kb * (1.0 - row)
            pend = pend * (1.0 - row)
            pend = jnp.where(lane == i, 0.0, pend)
            return kb, pend, nxt(pend)

        keep_blk, _, _ = lax.while_loop(
            cond, body, (keep_blk, pending, nxt(pending)))
        keep_ref[:, base:base + BLK] = keep_blk

        if bi + 1 < NBLK:
            kcol = keep_blk.reshape(BLK, 1)
            sup = jnp.max(over[:, BLK:] * kcol, axis=0, keepdims=True)
            keep_ref[:, base + BLK:] = keep_ref[:, base + BLK:] * (1.0 - sup)

    keep_out_ref[:, :] = keep_ref[:, :]


def _run_nms(b, valid_f):
    cols = [b[:, k:k + 1] for k in range(4)]
    rows = [b[:, k].reshape(1, NC) for k in range(4)]
    keep = pl.pallas_call(
        _nms_kernel,
        out_shape=jax.ShapeDtypeStruct((1, NC), jnp.float32),
        scratch_shapes=[
            pltpu.VMEM((1, NC), jnp.float32),
            pltpu.VMEM((BLK, BLK), jnp.float32),
        ],
    )(*cols, *rows, valid_f.reshape(1, NC))
    return keep[0]


def _delta_to_bases(b6, boxes):
    x1 = boxes[:, 0]; y1 = boxes[:, 1]; x2 = boxes[:, 2]; y2 = boxes[:, 3]
    dx = x2 - x1
    dy = y2 - y1
    midx = (x1 + x2) / 2.0 + b6[:, 0] * dx
    midy = (y1 + y2) / 2.0 + b6[:, 1] * dy
    firstx = b6[:, 2]; firsty = b6[:, 3]; secondx = b6[:, 4]; secondy = b6[:, 5]
    X1 = midx + firstx * dx
    Y1 = midy + firsty * dy
    X2 = midx + secondx * dx
    Y2 = midy + secondy * dy
    X3 = midx - secondx * dx
    Y3 = midy - secondy * dy
    X4 = midx - firstx * dx
    Y4 = midy - firsty * dy
    return jnp.stack((X1, Y1, X2, Y2, X3, Y3, X4, Y4, midx, midy), axis=-1)


FIN = 128


@functools.partial(
    pl.kernel,
    out_type=jax.ShapeDtypeStruct((16 * FIN,), jnp.float32),
    mesh=_sc_mesh,
    scratch_types=[
        pltpu.VMEM((NC,), jnp.float32),
        pltpu.VMEM((NC,), jnp.float32),
        pltpu.VMEM((NC,), jnp.int32),
        pltpu.VMEM((NC,), jnp.float32),
        pltpu.VMEM((NC,), jnp.float32),
        pltpu.VMEM((NC,), jnp.float32),
        pltpu.VMEM((NC,), jnp.float32),
        pltpu.VMEM((NC,), jnp.int32),
        pltpu.VMEM((N,), jnp.float32),
        pltpu.VMEM((FIN,), jnp.int32),
        pltpu.VMEM((7 * FIN,), jnp.float32),
        pltpu.VMEM((16 * FIN,), jnp.float32),
    ],
    compiler_params=pltpu.CompilerParams(needs_layout_passes=False,
                                         use_tc_tiling_on_sc=False),
)
def _finalize(keep_hbm, score_hbm, idx_hbm, x1_hbm, y1_hbm, x2_hbm, y2_hbm,
              b0_hbm, b1_hbm, b2_hbm, b3_hbm, b4_hbm, b5_hbm, b6_hbm,
              out_hbm,
              keep_v, fscore_v, fidx_v, fx1_v, fy1_v, fx2_v, fy2_v,
              order_v, tab_v, oi_v, bas_v, out_v):
    wid = lax.axis_index("s") * 2 + lax.axis_index("c")

    @pl.when(wid == 0)
    def _():
        pltpu.sync_copy(keep_hbm, keep_v)
        pltpu.sync_copy(score_hbm, fscore_v)
        pltpu.sync_copy(idx_hbm, fidx_v)
        pltpu.sync_copy(x1_hbm, fx1_v)
        pltpu.sync_copy(y1_hbm, fy1_v)
        pltpu.sync_copy(x2_hbm, fx2_v)
        pltpu.sync_copy(y2_hbm, fy2_v)

        def c1(i, acc):
            k = keep_v[pl.ds(i * 16, 16)] > 0.5
            return acc + jnp.sum(jnp.asarray(k, jnp.int32))
        K = plsc.parallel_loop(0, NC // 16, 1, unroll=8,
                               carry=jnp.int32(0))(c1)

        def c2(i, carry):
            nk, nu = carry
            k = keep_v[pl.ds(i * 16, 16)] > 0.5
            ki = jnp.asarray(k, jnp.int32)
            ui = 1 - ki
            posk = nk + plsc.cumsum(ki) - ki
            posu = K + nu + plsc.cumsum(ui) - ui
            cand = i * 16 + _I16()
            plsc.store_scatter(order_v, [posk], cand, mask=k)
            plsc.store_scatter(order_v, [posu], cand, mask=~k)
            return nk + jnp.sum(ki), nu + jnp.sum(ui)
        plsc.parallel_loop(0, NC // 16, 1, unroll=8,
                           carry=(jnp.int32(0), jnp.int32(0)))(c2)

        def c3(j):
            ordc = order_v[pl.ds(j * 16, 16)]
            rows = j * 16 + _I16()
            sc = plsc.load_gather(fscore_v, [ordc])
            out_v[pl.ds(15 * FIN + j * 16, 16)] = jnp.where(
                rows < K, sc, -1.0)
            oi_v[pl.ds(j * 16, 16)] = plsc.load_gather(fidx_v, [ordc])
            x1 = jnp.clip(plsc.load_gather(fx1_v, [ordc]), 0.0, IMG_W)
            y1 = jnp.clip(plsc.load_gather(fy1_v, [ordc]), 0.0, IMG_H)
            x2 = jnp.clip(plsc.load_gather(fx2_v, [ordc]), 0.0, IMG_W)
            y2 = jnp.clip(plsc.load_gather(fy2_v, [ordc]), 0.0, IMG_H)
            out_v[pl.ds(11 * FIN + j * 16, 16)] = x1
            out_v[pl.ds(12 * FIN + j * 16, 16)] = y1
            out_v[pl.ds(13 * FIN + j * 16, 16)] = x2
            out_v[pl.ds(14 * FIN + j * 16, 16)] = y2
        plsc.parallel_loop(0, FIN // 16, 1, unroll=4)(c3)

        for c, src in enumerate((b0_hbm, b1_hbm, b2_hbm, b3_hbm,
                                 b4_hbm, b5_hbm, b6_hbm)):
            pltpu.sync_copy(src, tab_v)

            def c4(j, c=c):
                oic = oi_v[pl.ds(j * 16, 16)]
                bas_v[pl.ds(c * FIN + j * 16, 16)] = plsc.load_gather(
                    tab_v, [oic])
            plsc.parallel_loop(0, FIN // 16, 1, unroll=4)(c4)

        def c5(j):
            x1 = out_v[pl.ds(11 * FIN + j * 16, 16)]
            y1 = out_v[pl.ds(12 * FIN + j * 16, 16)]
            x2 = out_v[pl.ds(13 * FIN + j * 16, 16)]
            y2 = out_v[pl.ds(14 * FIN + j * 16, 16)]
            b0 = bas_v[pl.ds(0 * FIN + j * 16, 16)]
            b1 = bas_v[pl.ds(1 * FIN + j * 16, 16)]
            b2 = bas_v[pl.ds(2 * FIN + j * 16, 16)]
            b3 = bas_v[pl.ds(3 * FIN + j * 16, 16)]
            b4 = bas_v[pl.ds(4 * FIN + j * 16, 16)]
            b5 = bas_v[pl.ds(5 * FIN + j * 16, 16)]
            b6 = bas_v[pl.ds(6 * FIN + j * 16, 16)]
            dx = x2 - x1
            dy = y2 - y1
            midx = (x1 + x2) / 2.0 + b0 * dx
            midy = (y1 + y2) / 2.0 + b1 * dy
            out_v[pl.ds(0 * FIN + j * 16, 16)] = midx + b2 * dx
            out_v[pl.ds(1 * FIN + j * 16, 16)] = midy + b3 * dy
            out_v[pl.ds(2 * FIN + j * 16, 16)] = midx + b4 * dx
            out_v[pl.ds(3 * FIN + j * 16, 16)] = midy + b5 * dy
            out_v[pl.ds(4 * FIN + j * 16, 16)] = midx - b4 * dx
            out_v[pl.ds(5 * FIN + j * 16, 16)] = midy - b5 * dy
            out_v[pl.ds(6 * FIN + j * 16, 16)] = midx - b2 * dx
            out_v[pl.ds(7 * FIN + j * 16, 16)] = midy - b3 * dy
            out_v[pl.ds(8 * FIN + j * 16, 16)] = midx
            out_v[pl.ds(9 * FIN + j * 16, 16)] = midy
            out_v[pl.ds(10 * FIN + j * 16, 16)] = dy + b6 * dy
        plsc.parallel_loop(0, FIN // 16, 1, unroll=4)(c5)

        pltpu.sync_copy(out_v, out_hbm)


def kernel(boxes, scores, bases):
    s = scores[:, 0]
    c_sc, c_idx, cx1, cy1, cx2, cy2 = _select_sort(
        s, boxes[:, 0], boxes[:, 1], boxes[:, 2], boxes[:, 3])
    bx = jnp.stack([
        jnp.clip(cx1, 0.0, IMG_W),
        jnp.clip(cy1, 0.0, IMG_H),
        jnp.clip(cx2, 0.0, IMG_W),
        jnp.clip(cy2, 0.0, IMG_H),
    ], axis=1)
    valid = c_sc > SCORE_THRESH

    keep_f = _run_nms(bx, valid.astype(jnp.float32))

    out_flat = _finalize(keep_f, c_sc, c_idx, cx1, cy1, cx2, cy2,
                         bases[:, 0], bases[:, 1], bases[:, 2], bases[:, 3],
                         bases[:, 4], bases[:, 5], bases[:, 6])
    return jnp.stack(
        [out_flat[c * FIN:c * FIN + TOPK_PER_IMAGE] for c in range(16)],
        axis=1)
